# baseline (device time: 91212 ns/iter reference)
import jax
import jax.numpy as jnp
from jax import lax
from jax.experimental import pallas as pl
from jax.experimental.pallas import tpu as pltpu

N_DEV = 8

_PARTS = (
    (0, 1344, (1, 3, 4), ("p", "b1", "b2")),
    (1344, 1344, (3, 4, 1), ("b1", "b2", "b0")),
    (2688, 1408, (4, 1, 3), ("b2", "p", "b1")),
)
_SEC = (0, 4, 6)


def kernel(x, k, Wp):
    B, S, C = x.shape
    T = k.shape[0]
    _, P = Wp.shape
    R = B * S

    def body(x_ref, k_ref, wp_ref, out_ref,
             p_ref, rs0, rs1, rs2, send_sems, recv_sems):
        rs_comm = (rs0, rs1, rs2)
        d = lax.axis_index("i")
        b0 = d & 1
        b1 = (d >> 1) & 1
        b2 = (d >> 2) & 1
        sels = {"b0": b0, "b1": b1, "b2": b2, "p": b0 ^ b1}

        bar = pltpu.get_barrier_semaphore()
        for m in (1, 3, 4):
            pl.semaphore_signal(bar, inc=1, device_id=(d ^ m,),
                                device_id_type=pl.DeviceIdType.MESH)

        geo = []
        for start, rows, masks, selnames in _PARTS:
            u = rows // 8
            s_bits = [sels[n] for n in selnames]
            bases = [0]
            for j in range(3):
                bases.append(bases[j] + s_bits[j] * (4 >> j))
            geo.append((start, u, masks, s_bits, bases))

        def rs_rdma(p, kk):
            start, u, masks, s_bits, bases = geo[p]
            ns = 4 >> kk
            send0 = (bases[kk] + (1 - s_bits[kk]) * ns) * u + start
            return pltpu.make_async_remote_copy(
                src_ref=p_ref.at[pl.ds(send0, ns * u)],
                dst_ref=rs_comm[p].at[pl.ds(_SEC[kk] * u, ns * u)],
                send_sem=send_sems.at[p, kk],
                recv_sem=recv_sems.at[p, kk],
                device_id=(d ^ masks[kk],),
                device_id_type=pl.DeviceIdType.MESH,
            )

        def rs_acc(p, kk):
            start, u, masks, s_bits, bases = geo[p]
            ns = 4 >> kk
            keep0 = (bases[kk] + s_bits[kk] * ns) * u + start
            nr = ns * u
            sec = _SEC[kk] * u
            p_ref[pl.ds(keep0, nr), :] = (
                p_ref[pl.ds(keep0, nr), :] + rs_comm[p][pl.ds(sec, nr), :])

        def ag_rdma(p, kk):
            start, u, masks, s_bits, bases = geo[p]
            ns = 4 >> kk
            held0 = (bases[kk] + s_bits[kk] * ns) * u + start
            return pltpu.make_async_remote_copy(
                src_ref=p_ref.at[pl.ds(held0, ns * u)],
                dst_ref=p_ref.at[pl.ds(held0, ns * u)],
                send_sem=send_sems.at[p, 5 - kk],
                recv_sem=recv_sems.at[p, 5 - kk],
                device_id=(d ^ masks[kk],),
                device_id_type=pl.DeviceIdType.MESH,
            )

        def ag_copy_recv(p, kk):
            start, u, masks, s_bits, bases = geo[p]
            ns = 4 >> kk
            r0 = (bases[kk] + (1 - s_bits[kk]) * ns) * u + start
            out_ref[pl.ds(r0, ns * u), :] = p_ref[pl.ds(r0, ns * u), :]

        xv = x_ref[...]
        conv = xv * k_ref[T - 1, :]
        for t in range(T - 1):
            sh = T - 1 - t
            shifted = jnp.concatenate(
                [jnp.zeros((B, sh, C), jnp.float32), xv[:, :S - sh, :]],
                axis=1)
            conv = conv + shifted * k_ref[t, :]
        a = conv / (1.0 + jnp.exp(-conv))
        av = a.reshape(R, C)
        wp = wp_ref[...]

        rdmas = [None, None, None]
        for p, (start, rows, masks, selnames) in enumerate(_PARTS):
            p_ref[start:start + rows, :] = jnp.dot(
                av[start:start + rows, :], wp,
                preferred_element_type=jnp.float32)
            if p == 0:
                pl.semaphore_wait(bar, 3)
            rdmas[p] = rs_rdma(p, 0)
            rdmas[p].start()

        for kk in range(3):
            for p in range(3):
                rdmas[p].wait()
                rs_acc(p, kk)
                nxt = rs_rdma(p, kk + 1) if kk < 2 else ag_rdma(p, 2)
                nxt.start()
                rdmas[p] = nxt

        for kk in (2, 1, 0):
            if kk == 2:
                for p in range(3):
                    start, u, masks, s_bits, bases = geo[p]
                    own0 = bases[3] * u + start
                    out_ref[pl.ds(own0, u), :] = p_ref[pl.ds(own0, u), :]
            for p in range(3):
                rdmas[p].wait()
                if kk > 0:
                    nxt = ag_rdma(p, kk - 1)
                    nxt.start()
                    rdmas[p] = nxt
                ag_copy_recv(p, kk)

    out = pl.pallas_call(
        body,
        out_shape=jax.ShapeDtypeStruct((R, P), jnp.float32),
        in_specs=[pl.BlockSpec(memory_space=pltpu.VMEM)] * 3,
        out_specs=pl.BlockSpec(memory_space=pltpu.VMEM),
        scratch_shapes=[
            pltpu.VMEM((R, P), jnp.float32),
            pltpu.VMEM((7 * (1344 // 8), P), jnp.float32),
            pltpu.VMEM((7 * (1344 // 8), P), jnp.float32),
            pltpu.VMEM((7 * (1408 // 8), P), jnp.float32),
            pltpu.SemaphoreType.DMA((3, 6)),
            pltpu.SemaphoreType.DMA((3, 6)),
        ],
        compiler_params=pltpu.CompilerParams(
            collective_id=0,
            vmem_limit_bytes=100 * 1024 * 1024,
        ),
    )(x, k, Wp)
    return out.reshape(B, S, P)


# device time: 79760 ns/iter; 1.1436x vs baseline; 1.1436x over previous
import jax
import jax.numpy as jnp
from jax import lax
from jax.experimental import pallas as pl
from jax.experimental.pallas import tpu as pltpu

N_DEV = 8

_PARTS = (
    (0, 1344, (1, 3, 4), ("p", "b1", "b2")),
    (1344, 1344, (3, 4, 1), ("b1", "b2", "b0")),
    (2688, 1408, (4, 1, 3), ("b2", "p", "b1")),
)

R0H, R0C, R1H, R1C, R2, A2, A1O, A1Z, A0O, A0Z, A0Y1, A0Y2 = range(12)
_RS_SEC = {R0H: 0, R0C: 2, R1H: 4, R1C: 5, R2: 6}
_F2 = (0, 1, 0)


def kernel(x, k, Wp):
    B, S, C = x.shape
    T = k.shape[0]
    _, P = Wp.shape
    R = B * S

    def body(x_ref, k_ref, wp_ref, out_ref,
             p_ref, rs0, rs1, rs2, fw0, fw1, fw2, send_sems, recv_sems):
        rs_comm = (rs0, rs1, rs2)
        fwd = (fw0, fw1, fw2)
        d = lax.axis_index("i")
        b0 = d & 1
        b1 = (d >> 1) & 1
        b2 = (d >> 2) & 1
        sels = {"b0": b0, "b1": b1, "b2": b2, "p": b0 ^ b1}

        bar = pltpu.get_barrier_semaphore()
        for m in (1, 3, 4):
            pl.semaphore_signal(bar, inc=1, device_id=(d ^ m,),
                                device_id_type=pl.DeviceIdType.MESH)

        geo = []
        D = []
        for p, (start, rows, masks, selnames) in enumerate(_PARTS):
            u = rows // 8
            s0, s1, s2 = (sels[n] for n in selnames)
            ma, mb, mc = masks
            slot_O = 4 * s0 + 2 * s1 + s2
            slot_Z = 4 * s0 + 2 * s1 + (1 - s2)
            slot_Y1 = 4 * s0 + 2 * (1 - s1) + s2
            slot_Y2 = 4 * s0 + 2 * (1 - s1) + (1 - s2)
            g = dict(start=start, u=u, s0=s0, s1=s1, s2=s2,
                     O=slot_O, Z=slot_Z, Y1=slot_Y1, Y2=slot_Y2)
            geo.append(g)

            spec = {
                R0H: (4 * (1 - s0) + 2 * (1 - s1), 2, ma),
                R0C: (4 * (1 - s0) + 2 * s1, 2, ma),
                R1H: (4 * s0 + 2 * (1 - s1) + (1 - s2), 1, mb),
                R1C: (4 * s0 + 2 * (1 - s1) + s2, 1, mb),
                R2: (4 * s0 + 2 * s1 + (1 - s2), 1, mc),
                A2: (slot_O, 1, mc),
                A1O: (slot_O, 1, mb),
                A1Z: (slot_Z, 1, mb),
                A0O: (slot_O, 1, ma),
                A0Z: (slot_Z, 1, ma),
                A0Y1: (slot_Y1, 1, ma),
                A0Y2: (slot_Y2, 1, ma),
            }
            _FWD_SEC = {A1Z: 0, A0Z: 0, A0Y1: 1, A0Y2: 2}
            dd = {}
            for mid, (slot, ns, mask) in spec.items():
                src = p_ref.at[pl.ds(slot * u + start, ns * u)]
                if mid in _RS_SEC:
                    dst = rs_comm[p].at[pl.ds(_RS_SEC[mid] * u, ns * u)]
                else:
                    dst = src
                    if mid in _FWD_SEC:
                        src = fwd[p].at[pl.ds(_FWD_SEC[mid] * u, u)]
                dd[mid] = pltpu.make_async_remote_copy(
                    src_ref=src, dst_ref=dst,
                    send_sem=send_sems.at[p, mid],
                    recv_sem=recv_sems.at[p, mid],
                    device_id=(d ^ mask,),
                    device_id_type=pl.DeviceIdType.MESH,
                )
            t2 = s2 ^ _F2[p]
            ag_in = {
                A2: slot_Z,
                A1O: slot_Y1,
                A1Z: slot_Y2,
                A0O: 4 * (1 - s0) + 2 * s1 + t2,
                A0Z: 4 * (1 - s0) + 2 * s1 + (1 - t2),
                A0Y1: 4 * (1 - s0) + 2 * (1 - s1) + t2,
                A0Y2: 4 * (1 - s0) + 2 * (1 - s1) + (1 - t2),
            }
            for mid, slot in ag_in.items():
                dst = p_ref.at[pl.ds(slot * u + start, u)]
                dd[(mid, "recv")] = pltpu.make_async_remote_copy(
                    src_ref=dst, dst_ref=dst,
                    send_sem=send_sems.at[p, mid],
                    recv_sem=recv_sems.at[p, mid],
                    device_id=(d ^ spec[mid][2],),
                    device_id_type=pl.DeviceIdType.MESH,
                )
            D.append(dd)

        def acc(p, slot, ns, sec):
            g = geo[p]
            r0 = slot * g["u"] + g["start"]
            nr = ns * g["u"]
            p_ref[pl.ds(r0, nr), :] = (
                p_ref[pl.ds(r0, nr), :]
                + rs_comm[p][pl.ds(sec * g["u"], nr), :])

        def copy_out(p, slot, ns):
            g = geo[p]
            r0 = slot * g["u"] + g["start"]
            nr = ns * g["u"]
            out_ref[pl.ds(r0, nr), :] = p_ref[pl.ds(r0, nr), :]

        xv = x_ref[...]
        conv = xv * k_ref[T - 1, :]
        for t in range(T - 1):
            sh = T - 1 - t
            shifted = jnp.concatenate(
                [jnp.zeros((B, sh, C), jnp.float32), xv[:, :S - sh, :]],
                axis=1)
            conv = conv + shifted * k_ref[t, :]
        a = conv / (1.0 + jnp.exp(-conv))
        p_ref[...] = jnp.dot(a.reshape(R, C), wp_ref[...],
                             preferred_element_type=jnp.float32)

        pl.semaphore_wait(bar, 3)

        for p in range(3):
            D[p][R0H].start()
            D[p][R0C].start()
        for p in range(3):
            g = geo[p]
            D[p][R0H].wait_recv()
            acc(p, 4 * g["s0"] + 2 * (1 - g["s1"]), 2, _RS_SEC[R0H])
            D[p][R1H].start()
            D[p][R1C].start()
        for p in range(3):
            g = geo[p]
            D[p][R0C].wait_recv()
            acc(p, 4 * g["s0"] + 2 * g["s1"], 2, _RS_SEC[R0C])
        for p in range(3):
            g = geo[p]
            D[p][R1H].wait_recv()
            acc(p, 4 * g["s0"] + 2 * g["s1"] + (1 - g["s2"]), 1, _RS_SEC[R1H])
            D[p][R2].start()
        for p in range(3):
            D[p][R1C].wait_recv()
            acc(p, geo[p]["O"], 1, _RS_SEC[R1C])
        for p in range(3):
            D[p][R2].wait_recv()
            acc(p, geo[p]["O"], 1, _RS_SEC[R2])
            D[p][A2].start()
            D[p][A1O].start()
            D[p][A0O].start()
        for p in range(3):
            copy_out(p, geo[p]["O"], 1)
        for p in range(3):
            g = geo[p]
            u = g["u"]
            D[p][(A2, "recv")].wait_recv()
            fwd[p][pl.ds(0, u), :] = p_ref[pl.ds(g["Z"] * u + g["start"], u), :]
            D[p][A1Z].start()
            D[p][A0Z].start()
        for p in range(3):
            copy_out(p, geo[p]["Z"], 1)
        for p in range(3):
            g = geo[p]
            u = g["u"]
            D[p][(A1O, "recv")].wait_recv()
            fwd[p][pl.ds(u, u), :] = p_ref[pl.ds(g["Y1"] * u + g["start"], u), :]
            D[p][A0Y1].start()
        for p in range(3):
            g = geo[p]
            u = g["u"]
            D[p][(A1Z, "recv")].wait_recv()
            fwd[p][pl.ds(2 * u, u), :] = p_ref[pl.ds(g["Y2"] * u + g["start"], u), :]
            D[p][A0Y2].start()
        for p in range(3):
            copy_out(p, geo[p]["Y1"], 1)
            copy_out(p, geo[p]["Y2"], 1)
        for p in range(3):
            for mid in (A0O, A0Z, A0Y1, A0Y2):
                D[p][(mid, "recv")].wait_recv()
        for p in range(3):
            copy_out(p, 4 * (1 - geo[p]["s0"]), 4)
        for p in range(3):
            for mid in range(12):
                D[p][mid].wait_send()

    out = pl.pallas_call(
        body,
        out_shape=jax.ShapeDtypeStruct((R, P), jnp.float32),
        in_specs=[pl.BlockSpec(memory_space=pltpu.VMEM)] * 3,
        out_specs=pl.BlockSpec(memory_space=pltpu.VMEM),
        scratch_shapes=[
            pltpu.VMEM((R, P), jnp.float32),
            pltpu.VMEM((7 * (1344 // 8), P), jnp.float32),
            pltpu.VMEM((7 * (1344 // 8), P), jnp.float32),
            pltpu.VMEM((7 * (1408 // 8), P), jnp.float32),
            pltpu.VMEM((3 * (1344 // 8), P), jnp.float32),
            pltpu.VMEM((3 * (1344 // 8), P), jnp.float32),
            pltpu.VMEM((3 * (1408 // 8), P), jnp.float32),
            pltpu.SemaphoreType.DMA((3, 12)),
            pltpu.SemaphoreType.DMA((3, 12)),
        ],
        compiler_params=pltpu.CompilerParams(
            collective_id=0,
            vmem_limit_bytes=100 * 1024 * 1024,
        ),
    )(x, k, Wp)
    return out.reshape(B, S, P)


# device time: 79536 ns/iter; 1.1468x vs baseline; 1.0028x over previous
import jax
import jax.numpy as jnp
from jax import lax
from jax.experimental import pallas as pl
from jax.experimental.pallas import tpu as pltpu

N_DEV = 8

_PARTS = (
    (0, 1344, (1, 3, 4), ("p", "b1", "b2")),
    (1344, 1344, (3, 4, 1), ("b1", "b2", "b0")),
    (2688, 1408, (4, 1, 3), ("b2", "p", "b1")),
)

R0H, R0C, R1H, R1C, R2, A2, A1O, A1Z, A0O, A0Z, A0Y1, A0Y2, R0C2 = range(13)
_RS_SEC = {R0H: 0, R0C: 2, R0C2: 3, R1H: 4, R1C: 5, R2: 6}
_F2 = (0, 1, 0)


def kernel(x, k, Wp):
    B, S, C = x.shape
    T = k.shape[0]
    _, P = Wp.shape
    R = B * S

    def body(x_ref, k_ref, wp_ref, out_ref,
             p_ref, rs0, rs1, rs2, fw0, fw1, fw2, send_sems, recv_sems):
        rs_comm = (rs0, rs1, rs2)
        fwd = (fw0, fw1, fw2)
        d = lax.axis_index("i")
        b0 = d & 1
        b1 = (d >> 1) & 1
        b2 = (d >> 2) & 1
        sels = {"b0": b0, "b1": b1, "b2": b2, "p": b0 ^ b1}

        bar = pltpu.get_barrier_semaphore()
        for m in (1, 3, 4):
            pl.semaphore_signal(bar, inc=1, device_id=(d ^ m,),
                                device_id_type=pl.DeviceIdType.MESH)

        geo = []
        D = []
        for p, (start, rows, masks, selnames) in enumerate(_PARTS):
            u = rows // 8
            s0, s1, s2 = (sels[n] for n in selnames)
            ma, mb, mc = masks
            slot_O = 4 * s0 + 2 * s1 + s2
            slot_Z = 4 * s0 + 2 * s1 + (1 - s2)
            slot_Y1 = 4 * s0 + 2 * (1 - s1) + s2
            slot_Y2 = 4 * s0 + 2 * (1 - s1) + (1 - s2)
            g = dict(start=start, u=u, s0=s0, s1=s1, s2=s2,
                     O=slot_O, Z=slot_Z, Y1=slot_Y1, Y2=slot_Y2)
            geo.append(g)

            t2 = s2 ^ _F2[p]
            spec = {
                R0H: (4 * (1 - s0) + 2 * (1 - s1), 2, ma),
                R0C: (4 * (1 - s0) + 2 * s1 + (1 - t2), 1, ma),
                R0C2: (4 * (1 - s0) + 2 * s1 + t2, 1, ma),
                R1H: (4 * s0 + 2 * (1 - s1) + (1 - s2), 1, mb),
                R1C: (4 * s0 + 2 * (1 - s1) + s2, 1, mb),
                R2: (4 * s0 + 2 * s1 + (1 - s2), 1, mc),
                A2: (slot_O, 1, mc),
                A1O: (slot_O, 1, mb),
                A1Z: (slot_Z, 1, mb),
                A0O: (slot_O, 1, ma),
                A0Z: (slot_Z, 1, ma),
                A0Y1: (slot_Y1, 1, ma),
                A0Y2: (slot_Y2, 1, ma),
            }
            _FWD_SEC = {A1Z: 0, A0Z: 0, A0Y1: 1, A0Y2: 2}
            dd = {}
            for mid, (slot, ns, mask) in spec.items():
                src = p_ref.at[pl.ds(slot * u + start, ns * u)]
                if mid in _RS_SEC:
                    dst = rs_comm[p].at[pl.ds(_RS_SEC[mid] * u, ns * u)]
                else:
                    dst = src
                    if mid in _FWD_SEC:
                        src = fwd[p].at[pl.ds(_FWD_SEC[mid] * u, u)]
                dd[mid] = pltpu.make_async_remote_copy(
                    src_ref=src, dst_ref=dst,
                    send_sem=send_sems.at[p, mid],
                    recv_sem=recv_sems.at[p, mid],
                    device_id=(d ^ mask,),
                    device_id_type=pl.DeviceIdType.MESH,
                )
            ag_in = {
                A2: slot_Z,
                A1O: slot_Y1,
                A1Z: slot_Y2,
                A0O: 4 * (1 - s0) + 2 * s1 + t2,
                A0Z: 4 * (1 - s0) + 2 * s1 + (1 - t2),
                A0Y1: 4 * (1 - s0) + 2 * (1 - s1) + t2,
                A0Y2: 4 * (1 - s0) + 2 * (1 - s1) + (1 - t2),
            }
            for mid, slot in ag_in.items():
                dst = p_ref.at[pl.ds(slot * u + start, u)]
                dd[(mid, "recv")] = pltpu.make_async_remote_copy(
                    src_ref=dst, dst_ref=dst,
                    send_sem=send_sems.at[p, mid],
                    recv_sem=recv_sems.at[p, mid],
                    device_id=(d ^ spec[mid][2],),
                    device_id_type=pl.DeviceIdType.MESH,
                )
            g["a0_in"] = {mid: ag_in[mid] for mid in (A0O, A0Z, A0Y1, A0Y2)}
            D.append(dd)

        def acc(p, slot, ns, sec):
            g = geo[p]
            r0 = slot * g["u"] + g["start"]
            nr = ns * g["u"]
            p_ref[pl.ds(r0, nr), :] = (
                p_ref[pl.ds(r0, nr), :]
                + rs_comm[p][pl.ds(sec * g["u"], nr), :])

        def copy_out(p, slot, ns):
            g = geo[p]
            r0 = slot * g["u"] + g["start"]
            nr = ns * g["u"]
            out_ref[pl.ds(r0, nr), :] = p_ref[pl.ds(r0, nr), :]

        xv = x_ref[...]
        conv = xv * k_ref[T - 1, :]
        for t in range(T - 1):
            sh = T - 1 - t
            shifted = jnp.concatenate(
                [jnp.zeros((B, sh, C), jnp.float32), xv[:, :S - sh, :]],
                axis=1)
            conv = conv + shifted * k_ref[t, :]
        a = conv / (1.0 + jnp.exp(-conv))
        p_ref[...] = jnp.dot(a.reshape(R, C), wp_ref[...],
                             preferred_element_type=jnp.float32)

        pl.semaphore_wait(bar, 3)

        for p in range(3):
            D[p][R0H].start()
            D[p][R0C].start()
            D[p][R0C2].start()
        for p in range(3):
            g = geo[p]
            D[p][R0H].wait_recv()
            acc(p, 4 * g["s0"] + 2 * (1 - g["s1"]), 2, _RS_SEC[R0H])
            D[p][R1H].start()
            D[p][R1C].start()
        for p in range(3):
            g = geo[p]
            D[p][R0C].wait_recv()
            acc(p, 4 * g["s0"] + 2 * g["s1"] + (1 - g["s2"]), 1, _RS_SEC[R0C])
        for p in range(3):
            g = geo[p]
            D[p][R1H].wait_recv()
            acc(p, 4 * g["s0"] + 2 * g["s1"] + (1 - g["s2"]), 1, _RS_SEC[R1H])
            D[p][R2].start()
        for p in range(3):
            D[p][R0C2].wait_recv()
            acc(p, geo[p]["O"], 1, _RS_SEC[R0C2])
        for p in range(3):
            D[p][R1C].wait_recv()
            acc(p, geo[p]["O"], 1, _RS_SEC[R1C])
        for p in range(3):
            D[p][R2].wait_recv()
            acc(p, geo[p]["O"], 1, _RS_SEC[R2])
            D[p][A2].start()
            D[p][A1O].start()
            D[p][A0O].start()
        for p in range(3):
            copy_out(p, geo[p]["O"], 1)
        for p in range(3):
            g = geo[p]
            u = g["u"]
            D[p][(A2, "recv")].wait_recv()
            fwd[p][pl.ds(0, u), :] = p_ref[pl.ds(g["Z"] * u + g["start"], u), :]
            D[p][A1Z].start()
            D[p][A0Z].start()
        for p in range(3):
            copy_out(p, geo[p]["Z"], 1)
        for p in range(3):
            g = geo[p]
            u = g["u"]
            D[p][(A1O, "recv")].wait_recv()
            fwd[p][pl.ds(u, u), :] = p_ref[pl.ds(g["Y1"] * u + g["start"], u), :]
            D[p][A0Y1].start()
        for p in range(3):
            g = geo[p]
            u = g["u"]
            D[p][(A1Z, "recv")].wait_recv()
            fwd[p][pl.ds(2 * u, u), :] = p_ref[pl.ds(g["Y2"] * u + g["start"], u), :]
            D[p][A0Y2].start()
        for p in range(3):
            copy_out(p, geo[p]["Y1"], 1)
            copy_out(p, geo[p]["Y2"], 1)
        for mid in (A0O, A0Z, A0Y1, A0Y2):
            for p in range(3):
                D[p][(mid, "recv")].wait_recv()
                copy_out(p, geo[p]["a0_in"][mid], 1)
        for p in range(3):
            for mid in range(13):
                D[p][mid].wait_send()

    out = pl.pallas_call(
        body,
        out_shape=jax.ShapeDtypeStruct((R, P), jnp.float32),
        in_specs=[pl.BlockSpec(memory_space=pltpu.VMEM)] * 3,
        out_specs=pl.BlockSpec(memory_space=pltpu.VMEM),
        scratch_shapes=[
            pltpu.VMEM((R, P), jnp.float32),
            pltpu.VMEM((7 * (1344 // 8), P), jnp.float32),
            pltpu.VMEM((7 * (1344 // 8), P), jnp.float32),
            pltpu.VMEM((7 * (1408 // 8), P), jnp.float32),
            pltpu.VMEM((3 * (1344 // 8), P), jnp.float32),
            pltpu.VMEM((3 * (1344 // 8), P), jnp.float32),
            pltpu.VMEM((3 * (1408 // 8), P), jnp.float32),
            pltpu.SemaphoreType.DMA((3, 13)),
            pltpu.SemaphoreType.DMA((3, 13)),
        ],
        compiler_params=pltpu.CompilerParams(
            collective_id=0,
            vmem_limit_bytes=100 * 1024 * 1024,
        ),
    )(x, k, Wp)
    return out.reshape(B, S, P)
